# baseline (speedup 1.0000x reference)
import functools

import jax
import jax.numpy as jnp
from jax import lax
from jax.experimental import pallas as pl
from jax.experimental.pallas import tpu as pltpu
from jax.experimental.pallas import tpu_sc as plsc

_N = 4194304
_NC = 2
_NS = 16
_NW = _NC * _NS
_PER_W = _N // _NW
_CH = 8192
_NBUF = 4
_NCH = _PER_W // _CH
_NGRP = _NCH // _NBUF
_L = 16
_U = 4

_mesh = plsc.VectorSubcoreMesh(core_axis_name="c", subcore_axis_name="s")


@functools.partial(
    pl.kernel,
    mesh=_mesh,
    out_type=jax.ShapeDtypeStruct((_N,), jnp.float32),
    scratch_types=(
        [pltpu.VMEM((_L,), jnp.float32)] * 2
        + [pltpu.VMEM((_CH,), jnp.float32)] * (2 * _NBUF)
        + [pltpu.SemaphoreType.DMA] * (2 * _NBUF)
    ),
)
def _sc_affine(x_hbm, s_hbm, t_hbm, o_hbm, *scratch):
    sv, tv = scratch[0], scratch[1]
    in_bufs = scratch[2:2 + _NBUF]
    out_bufs = scratch[2 + _NBUF:2 + 2 * _NBUF]
    in_sems = scratch[2 + 2 * _NBUF:2 + 3 * _NBUF]
    out_sems = scratch[2 + 3 * _NBUF:2 + 4 * _NBUF]

    wid = lax.axis_index("s") * _NC + lax.axis_index("c")
    base = wid * _PER_W

    pltpu.sync_copy(s_hbm, sv)
    pltpu.sync_copy(t_hbm, tv)
    svec = sv[...]
    tvec = tv[...]

    for b in range(_NBUF - 1):
        pltpu.async_copy(
            x_hbm.at[pl.ds(base + b * _CH, _CH)], in_bufs[b], in_sems[b])

    def group(g, carry):
        for b in range(_NBUF):
            c = g * _NBUF + b
            cn = c + _NBUF - 1
            sn = (b + _NBUF - 1) % _NBUF

            @pl.when(cn < _NCH)
            def _():
                pltpu.async_copy(
                    x_hbm.at[pl.ds(base + cn * _CH, _CH)],
                    in_bufs[sn], in_sems[sn])

            pltpu.make_async_copy(
                x_hbm.at[pl.ds(base, _CH)], in_bufs[b], in_sems[b]).wait()

            @pl.when(g > 0)
            def _():
                pltpu.make_async_copy(
                    x_hbm.at[pl.ds(base, _CH)], out_bufs[b],
                    out_sems[b]).wait()

            def inner(i, acc):
                for u in range(_U):
                    off = (i * _U + u) * _L
                    out_bufs[b][pl.ds(off, _L)] = (
                        in_bufs[b][pl.ds(off, _L)] * svec + tvec)
                return acc

            lax.fori_loop(0, _CH // (_L * _U), inner, 0)

            pltpu.async_copy(
                out_bufs[b], o_hbm.at[pl.ds(base + c * _CH, _CH)],
                out_sems[b])
        return carry

    lax.fori_loop(0, _NGRP, group, 0)

    for b in range(_NBUF):
        pltpu.make_async_copy(
            x_hbm.at[pl.ds(base, _CH)], out_bufs[b], out_sems[b]).wait()


def kernel(x, head, scale, shift):
    del head
    s16 = jnp.broadcast_to(scale.astype(jnp.float32), (_L,))
    t16 = jnp.broadcast_to(shift.astype(jnp.float32), (_L,))
    return _sc_affine(x, s16, t16)


# speedup vs baseline: 1.0163x; 1.0163x over previous
import functools

import jax
import jax.numpy as jnp
from jax import lax
from jax.experimental import pallas as pl
from jax.experimental.pallas import tpu as pltpu
from jax.experimental.pallas import tpu_sc as plsc

_N = 4194304
_NC = 2
_NS = 16
_NW = _NC * _NS
_PER_W = _N // _NW
_CH = 8192
_NBUF = 4
_NCH = _PER_W // _CH
_NGRP = _NCH // _NBUF
_L = 16
_U = 8

_mesh = plsc.VectorSubcoreMesh(core_axis_name="c", subcore_axis_name="s")


@functools.partial(
    pl.kernel,
    mesh=_mesh,
    out_type=jax.ShapeDtypeStruct((_N,), jnp.float32),
    scratch_types=(
        [pltpu.VMEM((_L,), jnp.float32)] * 2
        + [pltpu.VMEM((_CH,), jnp.float32)] * (2 * _NBUF)
        + [pltpu.SemaphoreType.DMA] * (2 * _NBUF)
    ),
)
def _sc_affine(x_hbm, s_hbm, t_hbm, o_hbm, *scratch):
    sv, tv = scratch[0], scratch[1]
    in_bufs = scratch[2:2 + _NBUF]
    out_bufs = scratch[2 + _NBUF:2 + 2 * _NBUF]
    in_sems = scratch[2 + 2 * _NBUF:2 + 3 * _NBUF]
    out_sems = scratch[2 + 3 * _NBUF:2 + 4 * _NBUF]

    wid = lax.axis_index("s") * _NC + lax.axis_index("c")
    base = wid * _PER_W

    pltpu.sync_copy(s_hbm, sv)
    pltpu.sync_copy(t_hbm, tv)
    svec = sv[...]
    tvec = tv[...]

    for b in range(_NBUF - 1):
        pltpu.async_copy(
            x_hbm.at[pl.ds(base + b * _CH, _CH)], in_bufs[b], in_sems[b])

    def group(g, carry):
        for b in range(_NBUF):
            c = g * _NBUF + b
            cn = c + _NBUF - 1
            sn = (b + _NBUF - 1) % _NBUF

            @pl.when(cn < _NCH)
            def _():
                pltpu.async_copy(
                    x_hbm.at[pl.ds(base + cn * _CH, _CH)],
                    in_bufs[sn], in_sems[sn])

            pltpu.make_async_copy(
                x_hbm.at[pl.ds(base, _CH)], in_bufs[b], in_sems[b]).wait()

            @pl.when(g > 0)
            def _():
                pltpu.make_async_copy(
                    x_hbm.at[pl.ds(base, _CH)], out_bufs[b],
                    out_sems[b]).wait()

            def inner(i, acc):
                for u in range(_U):
                    off = (i * _U + u) * _L
                    out_bufs[b][pl.ds(off, _L)] = (
                        in_bufs[b][pl.ds(off, _L)] * svec + tvec)
                return acc

            lax.fori_loop(0, _CH // (_L * _U), inner, 0)

            pltpu.async_copy(
                out_bufs[b], o_hbm.at[pl.ds(base + c * _CH, _CH)],
                out_sems[b])
        return carry

    lax.fori_loop(0, _NGRP, group, 0)

    for b in range(_NBUF):
        pltpu.make_async_copy(
            x_hbm.at[pl.ds(base, _CH)], out_bufs[b], out_sems[b]).wait()


def kernel(x, head, scale, shift):
    del head
    s16 = jnp.broadcast_to(scale.astype(jnp.float32), (_L,))
    t16 = jnp.broadcast_to(shift.astype(jnp.float32), (_L,))
    return _sc_affine(x, s16, t16)
